# Initial kernel scaffold; baseline (speedup 1.0000x reference)
import functools

import jax
import jax.numpy as jnp
from jax import lax
from jax.experimental import pallas as pl
from jax.experimental.pallas import tpu as pltpu
from jax.experimental.pallas import tpu_sc as plsc

N = 10000
E = 320000
D_IN = 128
D_OUT = 128
HALF = 64

NUM_SUB = 16
LANES = 16

EDGE_CHUNK = 80
EDGES_PER_SUB = E // NUM_SUB
ROWS_PER_SUB = N // NUM_SUB
BIAS_TILE = 125


def _matmul_halves(H, W):
    n, d_in = H.shape
    d_out = W.shape[1]
    bn = 2000

    def mm_body(h_ref, w_ref, o0_ref, o1_ref):
        hw = jnp.dot(h_ref[...], w_ref[...], preferred_element_type=jnp.float32)
        o0_ref[...] = hw[:, :HALF]
        o1_ref[...] = hw[:, HALF:]

    return pl.pallas_call(
        mm_body,
        grid=(n // bn,),
        in_specs=[
            pl.BlockSpec((bn, d_in), lambda i: (i, 0)),
            pl.BlockSpec((d_in, d_out), lambda i: (0, 0)),
        ],
        out_specs=[
            pl.BlockSpec((bn, HALF), lambda i: (i, 0)),
            pl.BlockSpec((bn, HALF), lambda i: (i, 0)),
        ],
        out_shape=[jax.ShapeDtypeStruct((n, HALF), jnp.float32)] * 2,
    )(H, W)


def _spmm_sc(hw0, hw1, src, dst, vals, b):
    mesh = plsc.VectorSubcoreMesh(core_axis_name="c", subcore_axis_name="s")

    @functools.partial(
        pl.kernel,
        out_type=[jax.ShapeDtypeStruct((N, HALF), jnp.float32)] * 2,
        mesh=mesh,
        scratch_types=[
            pltpu.VMEM((EDGE_CHUNK,), jnp.int32),
            pltpu.VMEM((EDGE_CHUNK,), jnp.int32),
            pltpu.VMEM((EDGE_CHUNK,), jnp.float32),
            pltpu.VMEM((EDGE_CHUNK, HALF), jnp.float32),
            pltpu.VMEM((BIAS_TILE, HALF), jnp.float32),
            pltpu.VMEM_SHARED((N, HALF), jnp.float32),
        ],
    )
    def sc_kernel(hw0_hbm, hw1_hbm, src_hbm, dst_hbm, val_hbm, b_hbm,
                  o0_hbm, o1_hbm,
                  src_v, dst_v, val_v, rows_v, btile_v, acc_sh):
        core = lax.axis_index("c")
        sub = lax.axis_index("s")

        def run(hw_hbm, o_hbm, col0):
            pltpu.sync_copy(b_hbm.at[pl.ds(col0, HALF)], btile_v.at[0])

            @pl.loop(1, BIAS_TILE)
            def _(r):
                for cb in range(HALF // LANES):
                    sl = pl.ds(cb * LANES, LANES)
                    btile_v[r, sl] = btile_v[0, sl]

            row0 = sub * ROWS_PER_SUB

            @pl.loop(0, ROWS_PER_SUB // BIAS_TILE)
            def _(t):
                pltpu.sync_copy(
                    btile_v, acc_sh.at[pl.ds(row0 + t * BIAS_TILE, BIAS_TILE)])

            plsc.subcore_barrier()

            e0 = sub * EDGES_PER_SUB

            @pl.loop(0, EDGES_PER_SUB // EDGE_CHUNK)
            def _(t):
                off = e0 + t * EDGE_CHUNK
                pltpu.sync_copy(src_hbm.at[pl.ds(off, EDGE_CHUNK)], src_v)
                pltpu.sync_copy(dst_hbm.at[pl.ds(off, EDGE_CHUNK)], dst_v)
                pltpu.sync_copy(val_hbm.at[pl.ds(off, EDGE_CHUNK)], val_v)
                pltpu.sync_copy(hw_hbm.at[src_v], rows_v)

                @pl.loop(0, EDGE_CHUNK)
                def _(i):
                    scale = plsc.load_gather(
                        val_v, [jnp.full((LANES,), i, jnp.int32)])
                    for cb in range(HALF // LANES):
                        sl = pl.ds(cb * LANES, LANES)
                        rows_v[i, sl] = rows_v[i, sl] * scale

                pltpu.sync_copy(rows_v, acc_sh.at[dst_v], add=True)

            plsc.subcore_barrier()

            pltpu.sync_copy(acc_sh.at[pl.ds(row0, ROWS_PER_SUB)],
                            o_hbm.at[pl.ds(row0, ROWS_PER_SUB)])

        @pl.when(core == 0)
        def _():
            run(hw0_hbm, o0_hbm, 0)

        @pl.when(core == 1)
        def _():
            run(hw1_hbm, o1_hbm, HALF)

    return sc_kernel(hw0, hw1, src, dst, vals, b)


def kernel(A_indices, A_values, H, W, b):
    hw0, hw1 = _matmul_halves(H, W)
    src = A_indices[1]
    dst = A_indices[0]
    o0, o1 = _spmm_sc(hw0, hw1, src, dst, A_values, b)
    return jnp.concatenate([o0, o1], axis=1)


# speedup vs baseline: 7.2489x; 7.2489x over previous
import dataclasses
import functools

import jax
import jax.numpy as jnp
from jax import lax
from jax.experimental import pallas as pl
from jax.experimental.pallas import tpu as pltpu
from jax.experimental.pallas import tpu_sc as plsc

N = 10000
E = 320000
D_IN = 128
D_OUT = 128
HALF = 64

NUM_SUB = 16
LANES = 16

EDGE_CHUNK = 80
EDGES_PER_SUB = E // NUM_SUB
CHUNKS = EDGES_PER_SUB // EDGE_CHUNK
ROWS_PER_SUB = 624
ROWS_TAIL = N - NUM_SUB * ROWS_PER_SUB
BIAS_TILE = 208


def _matmul_halves(H, W):
    n, d_in = H.shape
    d_out = W.shape[1]
    bn = 2000

    def mm_body(h_ref, w_ref, o0_ref, o1_ref):
        hw = jnp.dot(h_ref[...], w_ref[...], preferred_element_type=jnp.float32)
        o0_ref[...] = hw[:, :HALF]
        o1_ref[...] = hw[:, HALF:]

    return pl.pallas_call(
        mm_body,
        grid=(n // bn,),
        in_specs=[
            pl.BlockSpec((bn, d_in), lambda i: (i, 0)),
            pl.BlockSpec((d_in, d_out), lambda i: (0, 0)),
        ],
        out_specs=[
            pl.BlockSpec((bn, HALF), lambda i: (i, 0)),
            pl.BlockSpec((bn, HALF), lambda i: (i, 0)),
        ],
        out_shape=[jax.ShapeDtypeStruct((n, HALF), jnp.float32)] * 2,
    )(H, W)


def _spmm_sc(hw0, hw1, src, dst, vals, b):
    mesh = plsc.VectorSubcoreMesh(core_axis_name="c", subcore_axis_name="s")

    cp = pltpu.CompilerParams()
    if "needs_layout_passes" in pltpu.CompilerParams.__dataclass_fields__:
        cp = dataclasses.replace(cp, needs_layout_passes=False)
    if "use_tc_tiling_on_sc" in pltpu.CompilerParams.__dataclass_fields__:
        cp = dataclasses.replace(cp, use_tc_tiling_on_sc=False)

    @functools.partial(
        pl.kernel,
        out_type=jax.ShapeDtypeStruct((2, N, HALF), jnp.float32),
        mesh=mesh,
        compiler_params=cp,
        scratch_types=[
            pltpu.VMEM((CHUNKS, EDGE_CHUNK), jnp.int32),
            pltpu.VMEM((CHUNKS, EDGE_CHUNK), jnp.int32),
            pltpu.VMEM((EDGES_PER_SUB,), jnp.float32),
            pltpu.VMEM((EDGE_CHUNK, HALF), jnp.float32),
            pltpu.VMEM((EDGE_CHUNK, HALF), jnp.float32),
            pltpu.VMEM((BIAS_TILE, HALF), jnp.float32),
            pltpu.VMEM_SHARED((N, HALF), jnp.float32),
            pltpu.SemaphoreType.DMA,
            pltpu.SemaphoreType.DMA,
            pltpu.SemaphoreType.DMA,
        ],
    )
    def sc_kernel(hw0_hbm, hw1_hbm, src_hbm, dst_hbm, val_hbm, b_hbm,
                  o_hbm2,
                  src_v, dst_v, val_v, rows0_v, rows1_v, btile_v,
                  acc_sh, sem0, sem1, sem2):
        core = lax.axis_index("c")
        sub = lax.axis_index("s")

        def run(hw_hbm, o_hbm, col0):
            row0 = sub * ROWS_PER_SUB

            src_cp = pltpu.async_copy(src_hbm.at[sub], src_v, sem1)
            dst_cp = pltpu.async_copy(dst_hbm.at[sub], dst_v, sem2)
            val_cp = pltpu.async_copy(val_hbm.at[sub], val_v, sem1)

            pltpu.sync_copy(b_hbm.at[pl.ds(col0, HALF)], btile_v.at[0])

            @pl.loop(1, BIAS_TILE)
            def _(r):
                for cb in range(HALF // LANES):
                    sl = pl.ds(cb * LANES, LANES)
                    btile_v[r, sl] = btile_v[0, sl]

            @pl.loop(0, ROWS_PER_SUB // BIAS_TILE)
            def _(t):
                pltpu.sync_copy(
                    btile_v, acc_sh.at[pl.ds(row0 + t * BIAS_TILE, BIAS_TILE)])

            @pl.when(sub == 0)
            def _():
                pltpu.sync_copy(
                    btile_v.at[pl.ds(0, ROWS_TAIL)],
                    acc_sh.at[pl.ds(NUM_SUB * ROWS_PER_SUB, ROWS_TAIL)])

            src_cp.wait()
            dst_cp.wait()
            val_cp.wait()
            plsc.subcore_barrier()

            def start_gather(t, rows_ref, sem):
                return pltpu.async_copy(hw_hbm.at[src_v.at[t]], rows_ref, sem)

            def scale_and_scatter(t, rows_ref):
                base = t * EDGE_CHUNK

                @pl.loop(0, EDGE_CHUNK, step=4)
                def _(c):
                    for j in range(4):
                        i = c + j
                        scale = plsc.load_gather(
                            val_v, [jnp.full((LANES,), base + i, jnp.int32)])
                        for cb in range(HALF // LANES):
                            sl = pl.ds(cb * LANES, LANES)
                            rows_ref[i, sl] = rows_ref[i, sl] * scale

                pltpu.sync_copy(rows_ref, acc_sh.at[dst_v.at[t]], add=True)

            def wait_gather(rows_ref, sem):
                pltpu.make_async_copy(
                    hw_hbm.at[pl.ds(0, EDGE_CHUNK)], rows_ref, sem).wait()

            start_gather(0, rows0_v, sem0)

            @pl.loop(0, CHUNKS, step=2)
            def _(g):
                wait_gather(rows0_v, sem0)
                start_gather(g + 1, rows1_v, sem1)
                scale_and_scatter(g, rows0_v)

                wait_gather(rows1_v, sem1)
                nxt = jnp.minimum(g + 2, CHUNKS - 1)
                start_gather(nxt, rows0_v, sem0)
                scale_and_scatter(g + 1, rows1_v)

            wait_gather(rows0_v, sem0)

            plsc.subcore_barrier()

            pltpu.sync_copy(acc_sh.at[pl.ds(row0, ROWS_PER_SUB)],
                            o_hbm.at[pl.ds(row0, ROWS_PER_SUB)])

            @pl.when(sub == 0)
            def _():
                pltpu.sync_copy(
                    acc_sh.at[pl.ds(NUM_SUB * ROWS_PER_SUB, ROWS_TAIL)],
                    o_hbm.at[pl.ds(NUM_SUB * ROWS_PER_SUB, ROWS_TAIL)])

        @pl.when(core == 0)
        def _():
            run(hw0_hbm, o_hbm2.at[0], 0)

        @pl.when(core == 1)
        def _():
            run(hw1_hbm, o_hbm2.at[1], HALF)

    return sc_kernel(hw0, hw1, src, dst, vals, b)


def kernel(A_indices, A_values, H, W, b):
    hw0, hw1 = _matmul_halves(H, W)
    src = A_indices[1].reshape(NUM_SUB, CHUNKS, EDGE_CHUNK)
    dst = A_indices[0].reshape(NUM_SUB, CHUNKS, EDGE_CHUNK)
    vals = A_values.reshape(NUM_SUB, EDGES_PER_SUB)
    o2 = _spmm_sc(hw0, hw1, src, dst, vals, b)
    return jnp.concatenate([o2[0], o2[1]], axis=1)


# speedup vs baseline: 8.3179x; 1.1475x over previous
import dataclasses
import functools

import jax
import jax.numpy as jnp
from jax import lax
from jax.experimental import pallas as pl
from jax.experimental.pallas import tpu as pltpu
from jax.experimental.pallas import tpu_sc as plsc

N = 10000
E = 320000
D_IN = 128
D_OUT = 128
HALF = 64

NUM_SUB = 16
LANES = 16

EDGE_CHUNK = 80
EDGES_PER_SUB = E // NUM_SUB
CHUNKS = EDGES_PER_SUB // EDGE_CHUNK
ROWS_PER_SUB = 624
ROWS_TAIL = N - NUM_SUB * ROWS_PER_SUB
BIAS_TILE = 208


def _matmul_halves(H, W):
    n, d_in = H.shape
    d_out = W.shape[1]
    bn = 2000

    def mm_body(h_ref, w_ref, o0_ref, o1_ref):
        hw = jnp.dot(h_ref[...], w_ref[...], preferred_element_type=jnp.float32)
        o0_ref[...] = hw[:, :HALF]
        o1_ref[...] = hw[:, HALF:]

    return pl.pallas_call(
        mm_body,
        grid=(n // bn,),
        in_specs=[
            pl.BlockSpec((bn, d_in), lambda i: (i, 0)),
            pl.BlockSpec((d_in, d_out), lambda i: (0, 0)),
        ],
        out_specs=[
            pl.BlockSpec((bn, HALF), lambda i: (i, 0)),
            pl.BlockSpec((bn, HALF), lambda i: (i, 0)),
        ],
        out_shape=[jax.ShapeDtypeStruct((n, HALF), jnp.float32)] * 2,
    )(H, W)


def _spmm_sc(hw0, hw1, src, dst, vals, b):
    mesh = plsc.VectorSubcoreMesh(core_axis_name="c", subcore_axis_name="s")

    cp = pltpu.CompilerParams()
    if "needs_layout_passes" in pltpu.CompilerParams.__dataclass_fields__:
        cp = dataclasses.replace(cp, needs_layout_passes=False)
    if "use_tc_tiling_on_sc" in pltpu.CompilerParams.__dataclass_fields__:
        cp = dataclasses.replace(cp, use_tc_tiling_on_sc=False)

    @functools.partial(
        pl.kernel,
        out_type=jax.ShapeDtypeStruct((2, N, HALF), jnp.float32),
        mesh=mesh,
        compiler_params=cp,
        scratch_types=[
            pltpu.VMEM((CHUNKS, EDGE_CHUNK), jnp.int32),
            pltpu.VMEM((CHUNKS, EDGE_CHUNK), jnp.int32),
            pltpu.VMEM((EDGES_PER_SUB,), jnp.float32),
            pltpu.VMEM((EDGE_CHUNK, HALF), jnp.float32),
            pltpu.VMEM((EDGE_CHUNK, HALF), jnp.float32),
            pltpu.VMEM((BIAS_TILE, HALF), jnp.float32),
            pltpu.VMEM_SHARED((N, HALF), jnp.float32),
            pltpu.SemaphoreType.DMA,
            pltpu.SemaphoreType.DMA,
            pltpu.SemaphoreType.DMA,
            pltpu.SemaphoreType.DMA,
        ],
    )
    def sc_kernel(hw0_hbm, hw1_hbm, src_hbm, dst_hbm, val_hbm, b_hbm,
                  o_hbm2,
                  src_v, dst_v, val_v, rows0_v, rows1_v, btile_v,
                  acc_sh, sem0, sem1, sem2, sem3):
        core = lax.axis_index("c")
        sub = lax.axis_index("s")

        def run(hw_hbm, o_hbm, col0):
            row0 = sub * ROWS_PER_SUB

            src_cp = pltpu.async_copy(src_hbm.at[sub], src_v, sem1)
            dst_cp = pltpu.async_copy(dst_hbm.at[sub], dst_v, sem2)
            val_cp = pltpu.async_copy(val_hbm.at[sub], val_v, sem1)

            pltpu.sync_copy(b_hbm.at[pl.ds(col0, HALF)], btile_v.at[0])

            @pl.loop(1, BIAS_TILE)
            def _(r):
                for cb in range(HALF // LANES):
                    sl = pl.ds(cb * LANES, LANES)
                    btile_v[r, sl] = btile_v[0, sl]

            @pl.loop(0, ROWS_PER_SUB // BIAS_TILE)
            def _(t):
                pltpu.sync_copy(
                    btile_v, acc_sh.at[pl.ds(row0 + t * BIAS_TILE, BIAS_TILE)])

            @pl.when(sub == 0)
            def _():
                pltpu.sync_copy(
                    btile_v.at[pl.ds(0, ROWS_TAIL)],
                    acc_sh.at[pl.ds(NUM_SUB * ROWS_PER_SUB, ROWS_TAIL)])

            src_cp.wait()
            dst_cp.wait()
            val_cp.wait()
            plsc.subcore_barrier()

            def start_gather(t, rows_ref, sem):
                return pltpu.async_copy(hw_hbm.at[src_v.at[t]], rows_ref, sem)

            def scale(t, rows_ref):
                base = t * EDGE_CHUNK

                @plsc.parallel_loop(0, EDGE_CHUNK, unroll=8)
                def _(i):
                    sv = plsc.load_gather(
                        val_v, [jnp.full((LANES,), base + i, jnp.int32)])
                    for cb in range(HALF // LANES):
                        sl = pl.ds(cb * LANES, LANES)
                        rows_ref[i, sl] = rows_ref[i, sl] * sv

            def start_scatter(t, rows_ref, sem):
                return pltpu.async_copy(rows_ref, acc_sh.at[dst_v.at[t]], sem,
                                        add=True)

            def wait_gather(rows_ref, sem):
                pltpu.make_async_copy(
                    hw_hbm.at[pl.ds(0, EDGE_CHUNK)], rows_ref, sem).wait()

            start_gather(0, rows0_v, sem0)
            start_gather(1, rows1_v, sem1)

            @pl.loop(0, CHUNKS, step=2)
            def _(g):
                wait_gather(rows0_v, sem0)
                scale(g, rows0_v)
                sc0 = start_scatter(g, rows0_v, sem2)

                wait_gather(rows1_v, sem1)
                scale(g + 1, rows1_v)
                sc1 = start_scatter(g + 1, rows1_v, sem3)

                sc0.wait()
                start_gather(jnp.minimum(g + 2, CHUNKS - 1), rows0_v, sem0)
                sc1.wait()
                start_gather(jnp.minimum(g + 3, CHUNKS - 1), rows1_v, sem1)

            wait_gather(rows0_v, sem0)
            wait_gather(rows1_v, sem1)

            plsc.subcore_barrier()

            pltpu.sync_copy(acc_sh.at[pl.ds(row0, ROWS_PER_SUB)],
                            o_hbm.at[pl.ds(row0, ROWS_PER_SUB)])

            @pl.when(sub == 0)
            def _():
                pltpu.sync_copy(
                    acc_sh.at[pl.ds(NUM_SUB * ROWS_PER_SUB, ROWS_TAIL)],
                    o_hbm.at[pl.ds(NUM_SUB * ROWS_PER_SUB, ROWS_TAIL)])

        @pl.when(core == 0)
        def _():
            run(hw0_hbm, o_hbm2.at[0], 0)

        @pl.when(core == 1)
        def _():
            run(hw1_hbm, o_hbm2.at[1], HALF)

    return sc_kernel(hw0, hw1, src, dst, vals, b)


def kernel(A_indices, A_values, H, W, b):
    hw0, hw1 = _matmul_halves(H, W)
    src = A_indices[1].reshape(NUM_SUB, CHUNKS, EDGE_CHUNK)
    dst = A_indices[0].reshape(NUM_SUB, CHUNKS, EDGE_CHUNK)
    vals = A_values.reshape(NUM_SUB, EDGES_PER_SUB)
    o2 = _spmm_sc(hw0, hw1, src, dst, vals, b)
    return jnp.concatenate([o2[0], o2[1]], axis=1)
